# baseline (speedup 1.0000x reference)
import functools

import jax
import jax.numpy as jnp
from jax import lax
from jax.experimental import pallas as pl
from jax.experimental.pallas import tpu as pltpu
from jax.experimental.pallas import tpu_sc as plsc

_B = 16384
_D = 64
_NW = 32
_BPW = _B // _NW
_CH = 128
_NCH = _BPW // _CH

_BS = 1024


def _sc_gather_body(uidx_hbm, gidx_hbm, ptab_hbm, gtab_hbm,
                    x_out, uidx_v, gidx_v, urows_v, grows_v, sem_u, sem_g):
    wid = lax.axis_index("s") * 2 + lax.axis_index("c")
    base = wid * _BPW
    pltpu.sync_copy(uidx_hbm.at[wid], uidx_v)
    pltpu.sync_copy(gidx_hbm.at[wid], gidx_v)
    copies = []
    for j in range(_NCH):
        copies.append(pltpu.async_copy(
            ptab_hbm.at[uidx_v.at[j]], urows_v.at[pl.ds(j * _CH, _CH)], sem_u))
        copies.append(pltpu.async_copy(
            gtab_hbm.at[gidx_v.at[j]], grows_v.at[pl.ds(j * _CH, _CH)], sem_g))
    for c in copies:
        c.wait()
    pltpu.sync_copy(urows_v, x_out.at[pl.ds(base, _BPW), pl.ds(0, _D)])
    pltpu.sync_copy(grows_v, x_out.at[pl.ds(base, _BPW), pl.ds(_D, _D)])


@functools.cache
def _make_sc_gather():
    return functools.partial(
        pl.kernel,
        mesh=plsc.VectorSubcoreMesh(core_axis_name="c", subcore_axis_name="s"),
        compiler_params=pltpu.CompilerParams(use_tc_tiling_on_sc=False),
        out_type=jax.ShapeDtypeStruct((_B, 2 * _D), jnp.float32),
        scratch_types=[
            pltpu.VMEM((_NCH, _CH), jnp.int32),
            pltpu.VMEM((_NCH, _CH), jnp.int32),
            pltpu.VMEM((_BPW, _D), jnp.float32),
            pltpu.VMEM((_BPW, _D), jnp.float32),
            pltpu.SemaphoreType.DMA,
            pltpu.SemaphoreType.DMA,
        ],
    )(_sc_gather_body)


def _mlp_body(x_ref, w1_ref, b1_ref, w2_ref, b2_ref,
              w3_ref, b3_ref, w4_ref, b4_ref, o_ref):
    f32 = jnp.float32
    h = jnp.maximum(
        jnp.dot(x_ref[...], w1_ref[...], preferred_element_type=f32)
        + b1_ref[...], 0.0)
    h = jnp.maximum(
        jnp.dot(h, w2_ref[...], preferred_element_type=f32) + b2_ref[...], 0.0)
    h = jnp.maximum(
        jnp.dot(h, w3_ref[...], preferred_element_type=f32) + b3_ref[...], 0.0)
    o_ref[...] = jnp.dot(h, w4_ref[...], preferred_element_type=f32) + b4_ref[...]


def _mlp(x, w1, b1r, w2, b2r, w3, b3r, w4, b4r):
    full = lambda shape: pl.BlockSpec(shape, lambda i: (0, 0))
    return pl.pallas_call(
        _mlp_body,
        grid=(_B // _BS,),
        in_specs=[
            pl.BlockSpec((_BS, 2 * _D), lambda i: (i, 0)),
            full(w1.shape), full(b1r.shape),
            full(w2.shape), full(b2r.shape),
            full(w3.shape), full(b3r.shape),
            full(w4.shape), full(b4r.shape),
        ],
        out_specs=pl.BlockSpec((_BS, 1), lambda i: (i, 0)),
        out_shape=jax.ShapeDtypeStruct((_B, 1), jnp.float32),
    )(x, w1, b1r, w2, b2r, w3, b3r, w4, b4r)


def kernel(user, game, player_table, game_table, W1, b1, W2, b2, W3, b3, W4, b4):
    uidx = user.reshape(_NW, _NCH, _CH)
    gidx = game.reshape(_NW, _NCH, _CH)
    x = _make_sc_gather()(uidx, gidx, player_table, game_table)
    return _mlp(x, W1, b1.reshape(1, -1),
                W2, b2.reshape(1, -1),
                W3, b3.reshape(1, -1),
                W4, b4.reshape(1, 1))


# speedup vs baseline: 2.2221x; 2.2221x over previous
import functools

import jax
import jax.numpy as jnp
from jax import lax
from jax.experimental import pallas as pl
from jax.experimental.pallas import tpu as pltpu
from jax.experimental.pallas import tpu_sc as plsc

_B = 16384
_D = 64
_NW = 32
_BPW = _B // _NW
_CH = 64
_NCH = _BPW // _CH
_L = 16

_BS = 1024


def _gather_table(idx_hbm, tab3_hbm, out_hbm, base, idx_v, rows_v, sem):
    pltpu.sync_copy(idx_hbm.at[pl.ds(base, _BPW)], idx_v)
    lane = lax.iota(jnp.int32, _L)

    def body(k, carry):
        copies = []
        for g in range(_CH // _L):
            off = k * _CH + g * _L
            i16 = idx_v[pl.ds(off, _L)]
            t16 = lax.shift_right_logical(i16, jnp.int32(3))
            s16 = lax.bitwise_and(i16, jnp.int32(7))
            for l in range(_L):
                t = jnp.sum(jnp.where(lane == l, t16, 0))
                s = jnp.sum(jnp.where(lane == l, s16, 0))
                copies.append(pltpu.async_copy(
                    tab3_hbm.at[t, pl.ds(s, 1), :],
                    rows_v.at[pl.ds(off + l, 1), :], sem))
        for c in copies:
            c.wait()
        return carry

    lax.fori_loop(0, _NCH, body, jnp.int32(0))
    pltpu.sync_copy(rows_v, out_hbm.at[pl.ds(base, _BPW), :])


def _sc_gather_body(uidx_hbm, gidx_hbm, ptab3_hbm, gtab3_hbm,
                    u_out, g_out,
                    idx_v, rows_v, sem):
    wid = lax.axis_index("s") * 2 + lax.axis_index("c")
    base = wid * _BPW
    _gather_table(uidx_hbm, ptab3_hbm, u_out, base, idx_v, rows_v, sem)
    _gather_table(gidx_hbm, gtab3_hbm, g_out, base, idx_v, rows_v, sem)


@functools.cache
def _make_sc_gather():
    return functools.partial(
        pl.kernel,
        mesh=plsc.VectorSubcoreMesh(core_axis_name="c", subcore_axis_name="s"),
        compiler_params=pltpu.CompilerParams(needs_layout_passes=False),
        out_type=[
            jax.ShapeDtypeStruct((_B, _D), jnp.float32),
            jax.ShapeDtypeStruct((_B, _D), jnp.float32),
        ],
        scratch_types=[
            pltpu.VMEM((_BPW,), jnp.int32),
            pltpu.VMEM((_BPW, _D), jnp.float32),
            pltpu.SemaphoreType.DMA,
        ],
    )(_sc_gather_body)


def _mlp_body(u_ref, g_ref, w1a_ref, w1b_ref, b1_ref, w2_ref, b2_ref,
              w3_ref, b3_ref, w4_ref, b4_ref, o_ref):
    f32 = jnp.float32
    h = jnp.maximum(
        jnp.dot(u_ref[...], w1a_ref[...], preferred_element_type=f32)
        + jnp.dot(g_ref[...], w1b_ref[...], preferred_element_type=f32)
        + b1_ref[...], 0.0)
    h = jnp.maximum(
        jnp.dot(h, w2_ref[...], preferred_element_type=f32) + b2_ref[...], 0.0)
    h = jnp.maximum(
        jnp.dot(h, w3_ref[...], preferred_element_type=f32) + b3_ref[...], 0.0)
    o_ref[...] = jnp.dot(h, w4_ref[...], preferred_element_type=f32) + b4_ref[...]


def _mlp(u_emb, g_emb, w1a, w1b, b1r, w2, b2r, w3, b3r, w4, b4r):
    full = lambda shape: pl.BlockSpec(shape, lambda i: (0, 0))
    return pl.pallas_call(
        _mlp_body,
        grid=(_B // _BS,),
        in_specs=[
            pl.BlockSpec((_BS, _D), lambda i: (i, 0)),
            pl.BlockSpec((_BS, _D), lambda i: (i, 0)),
            full(w1a.shape), full(w1b.shape), full(b1r.shape),
            full(w2.shape), full(b2r.shape),
            full(w3.shape), full(b3r.shape),
            full(w4.shape), full(b4r.shape),
        ],
        out_specs=pl.BlockSpec((_BS, 1), lambda i: (i, 0)),
        out_shape=jax.ShapeDtypeStruct((_B, 1), jnp.float32),
    )(u_emb, g_emb, w1a, w1b, b1r, w2, b2r, w3, b3r, w4, b4r)


def kernel(user, game, player_table, game_table, W1, b1, W2, b2, W3, b3, W4, b4):
    uidx = user.reshape(_B)
    gidx = game.reshape(_B)
    ptab3 = player_table.reshape(player_table.shape[0] // 8, 8, _D)
    gtab3 = game_table.reshape(game_table.shape[0] // 8, 8, _D)
    u_emb, g_emb = _make_sc_gather()(uidx, gidx, ptab3, gtab3)
    return _mlp(u_emb, g_emb,
                W1[:_D], W1[_D:], b1.reshape(1, -1),
                W2, b2.reshape(1, -1),
                W3, b3.reshape(1, -1),
                W4, b4.reshape(1, 1))
